# baseline (device time: 160292 ns/iter reference)
import jax
import jax.numpy as jnp
from jax import lax
from jax.experimental import pallas as pl
from jax.experimental.pallas import tpu as pltpu

N_DEV = 4
CHUNK = 256
F = 4096


def kernel(x, dy):
    partial = lax.dot_general(
        x, dy, (((0,), (0,)), ((), ())), preferred_element_type=jnp.float32
    )

    def body(p_ref, out_ref, comm_ref, send_sems, recv_sems):
        my_x = lax.axis_index("x")
        my_y = lax.axis_index("y")
        my_z = lax.axis_index("z")
        left = (my_z + N_DEV - 1) % N_DEV
        right = (my_z + 1) % N_DEV

        barrier_sem = pltpu.get_barrier_semaphore()
        for nbr in (left, right):
            pl.semaphore_signal(
                barrier_sem,
                inc=1,
                device_id=(my_x, my_y, nbr),
                device_id_type=pl.DeviceIdType.MESH,
            )
        pl.semaphore_wait(barrier_sem, 2)

        j0 = (my_z + N_DEV - 1) % N_DEV
        comm_ref[0] = p_ref[pl.ds(j0 * CHUNK, CHUNK), :]

        for s in range(N_DEV - 1):
            rdma = pltpu.make_async_remote_copy(
                src_ref=comm_ref.at[s],
                dst_ref=comm_ref.at[s + 1],
                send_sem=send_sems.at[s],
                recv_sem=recv_sems.at[s],
                device_id=(my_x, my_y, right),
                device_id_type=pl.DeviceIdType.MESH,
            )
            rdma.start()
            rdma.wait()
            j = (my_z + 2 * N_DEV - 2 - s) % N_DEV
            if s < N_DEV - 2:
                comm_ref[s + 1] = comm_ref[s + 1] + p_ref[pl.ds(j * CHUNK, CHUNK), :]
            else:
                out_ref[...] = comm_ref[s + 1] + p_ref[pl.ds(j * CHUNK, CHUNK), :]

    return pl.pallas_call(
        body,
        out_shape=jax.ShapeDtypeStruct((CHUNK, F), jnp.float32),
        in_specs=[pl.BlockSpec(memory_space=pltpu.VMEM)],
        out_specs=pl.BlockSpec(memory_space=pltpu.VMEM),
        scratch_shapes=[
            pltpu.VMEM((N_DEV, CHUNK, F), jnp.float32),
            pltpu.SemaphoreType.DMA((N_DEV - 1,)),
            pltpu.SemaphoreType.DMA((N_DEV - 1,)),
        ],
        compiler_params=pltpu.CompilerParams(collective_id=0),
    )(partial)


# device time: 83743 ns/iter; 1.9141x vs baseline; 1.9141x over previous
import jax
import jax.numpy as jnp
from jax import lax
from jax.experimental import pallas as pl
from jax.experimental.pallas import tpu as pltpu

NZ = 4
NP = 8
CHUNK = 256
F = 4096
W = F // NP


def _ring_xy(r):
    rx = jnp.where(r < 4, 0, 1)
    ry = jnp.where(r < 4, r, 7 - r)
    return rx, ry


def kernel(x, dy):
    my_x = lax.axis_index("x")
    my_y = lax.axis_index("y")
    my_r = jnp.where(my_x == 0, my_y, 7 - my_y)

    dy_slice = lax.dynamic_slice(dy, (0, my_r * W), (dy.shape[0], W))
    partial = lax.dot_general(
        x, dy_slice, (((0,), (0,)), ((), ())), preferred_element_type=jnp.float32
    )

    def body(p_ref, out_ref, comm_rs, agbuf,
             rs_send, rs_recv, ag_send, ag_recv):
        my_x = lax.axis_index("x")
        my_y = lax.axis_index("y")
        my_z = lax.axis_index("z")
        my_r = jnp.where(my_x == 0, my_y, 7 - my_y)

        z_left = (my_z + NZ - 1) % NZ
        z_right = (my_z + 1) % NZ
        rl_x, rl_y = _ring_xy((my_r + NP - 1) % NP)
        rr_x, rr_y = _ring_xy((my_r + 1) % NP)

        barrier_sem = pltpu.get_barrier_semaphore()
        for dev in (
            (my_x, my_y, z_left),
            (my_x, my_y, z_right),
            (rl_x, rl_y, my_z),
            (rr_x, rr_y, my_z),
        ):
            pl.semaphore_signal(
                barrier_sem, inc=1, device_id=dev,
                device_id_type=pl.DeviceIdType.MESH,
            )
        pl.semaphore_wait(barrier_sem, 4)

        j0 = (my_z + NZ - 1) % NZ
        comm_rs[0] = p_ref[pl.ds(j0 * CHUNK, CHUNK), :]
        for s in range(NZ - 1):
            rdma = pltpu.make_async_remote_copy(
                src_ref=comm_rs.at[s],
                dst_ref=comm_rs.at[s + 1],
                send_sem=rs_send.at[s],
                recv_sem=rs_recv.at[s],
                device_id=(my_x, my_y, z_right),
                device_id_type=pl.DeviceIdType.MESH,
            )
            rdma.start()
            rdma.wait()
            j = (my_z + 2 * NZ - 2 - s) % NZ
            if s < NZ - 2:
                comm_rs[s + 1] = comm_rs[s + 1] + p_ref[pl.ds(j * CHUNK, CHUNK), :]
            else:
                agbuf[pl.ds(my_r, 1)] = (
                    comm_rs[s + 1] + p_ref[pl.ds(j * CHUNK, CHUNK), :]
                )[jnp.newaxis]

        for h in range(NP - 1):
            o = (my_r + NP - h) % NP if h else my_r
            rdma = pltpu.make_async_remote_copy(
                src_ref=agbuf.at[o],
                dst_ref=agbuf.at[o],
                send_sem=ag_send.at[h],
                recv_sem=ag_recv.at[h],
                device_id=(rr_x, rr_y, my_z),
                device_id_type=pl.DeviceIdType.MESH,
            )
            rdma.start()
            rdma.wait()

        for j in range(NP):
            out_ref[:, j * W:(j + 1) * W] = agbuf[j]

    return pl.pallas_call(
        body,
        out_shape=jax.ShapeDtypeStruct((CHUNK, F), jnp.float32),
        in_specs=[pl.BlockSpec(memory_space=pltpu.VMEM)],
        out_specs=pl.BlockSpec(memory_space=pltpu.VMEM),
        scratch_shapes=[
            pltpu.VMEM((NZ, CHUNK, W), jnp.float32),
            pltpu.VMEM((NP, CHUNK, W), jnp.float32),
            pltpu.SemaphoreType.DMA((NZ - 1,)),
            pltpu.SemaphoreType.DMA((NZ - 1,)),
            pltpu.SemaphoreType.DMA((NP - 1,)),
            pltpu.SemaphoreType.DMA((NP - 1,)),
        ],
        compiler_params=pltpu.CompilerParams(collective_id=0),
    )(partial)


# device time: 63806 ns/iter; 2.5122x vs baseline; 1.3125x over previous
import jax
import jax.numpy as jnp
from jax import lax
from jax.experimental import pallas as pl
from jax.experimental.pallas import tpu as pltpu

NZ = 4
NP = 8
CHUNK = 256
F = 4096
W = F // NP


def _ring_xy(r):
    rx = jnp.where(r < 4, 0, 1)
    ry = jnp.where(r < 4, r, 7 - r)
    return rx, ry


def kernel(x, dy):
    my_x = lax.axis_index("x")
    my_y = lax.axis_index("y")
    my_r = jnp.where(my_x == 0, my_y, 7 - my_y)

    dy_slice = lax.dynamic_slice(dy, (0, my_r * W), (dy.shape[0], W))
    partial = lax.dot_general(
        x, dy_slice, (((0,), (0,)), ((), ())), preferred_element_type=jnp.float32
    )

    def body(p_ref, out_ref, comm_rs, agbuf,
             rs_send, rs_recv, ag_send, ag_recv):
        my_x = lax.axis_index("x")
        my_y = lax.axis_index("y")
        my_z = lax.axis_index("z")
        my_r = jnp.where(my_x == 0, my_y, 7 - my_y)

        z_left = (my_z + NZ - 1) % NZ
        z_right = (my_z + 1) % NZ
        rl_x, rl_y = _ring_xy((my_r + NP - 1) % NP)
        rr_x, rr_y = _ring_xy((my_r + 1) % NP)

        barrier_sem = pltpu.get_barrier_semaphore()
        for dev in (
            (my_x, my_y, z_left),
            (my_x, my_y, z_right),
            (rl_x, rl_y, my_z),
            (rr_x, rr_y, my_z),
        ):
            pl.semaphore_signal(
                barrier_sem, inc=1, device_id=dev,
                device_id_type=pl.DeviceIdType.MESH,
            )
        pl.semaphore_wait(barrier_sem, 4)

        j0 = (my_z + NZ - 1) % NZ
        comm_rs[0] = p_ref[pl.ds(j0 * CHUNK, CHUNK), :]
        for s in range(NZ - 1):
            rdma = pltpu.make_async_remote_copy(
                src_ref=comm_rs.at[s],
                dst_ref=comm_rs.at[s + 1],
                send_sem=rs_send.at[s],
                recv_sem=rs_recv.at[s],
                device_id=(my_x, my_y, z_right),
                device_id_type=pl.DeviceIdType.MESH,
            )
            rdma.start()
            rdma.wait()
            j = (my_z + 2 * NZ - 2 - s) % NZ
            if s < NZ - 2:
                comm_rs[s + 1] = comm_rs[s + 1] + p_ref[pl.ds(j * CHUNK, CHUNK), :]
            else:
                agbuf[pl.ds(my_r, 1)] = (
                    comm_rs[s + 1] + p_ref[pl.ds(j * CHUNK, CHUNK), :]
                )[jnp.newaxis]

        for h in range(4):
            o_cw = (my_r + NP - h) % NP
            cw = pltpu.make_async_remote_copy(
                src_ref=agbuf.at[o_cw],
                dst_ref=agbuf.at[o_cw],
                send_sem=ag_send.at[h],
                recv_sem=ag_recv.at[h],
                device_id=(rr_x, rr_y, my_z),
                device_id_type=pl.DeviceIdType.MESH,
            )
            cw.start()
            if h < 3:
                o_ccw = (my_r + h) % NP
                ccw = pltpu.make_async_remote_copy(
                    src_ref=agbuf.at[o_ccw],
                    dst_ref=agbuf.at[o_ccw],
                    send_sem=ag_send.at[4 + h],
                    recv_sem=ag_recv.at[4 + h],
                    device_id=(rl_x, rl_y, my_z),
                    device_id_type=pl.DeviceIdType.MESH,
                )
                ccw.start()
                ccw.wait()
            cw.wait()

        for j in range(NP):
            out_ref[:, j * W:(j + 1) * W] = agbuf[j]

    return pl.pallas_call(
        body,
        out_shape=jax.ShapeDtypeStruct((CHUNK, F), jnp.float32),
        in_specs=[pl.BlockSpec(memory_space=pltpu.VMEM)],
        out_specs=pl.BlockSpec(memory_space=pltpu.VMEM),
        scratch_shapes=[
            pltpu.VMEM((NZ, CHUNK, W), jnp.float32),
            pltpu.VMEM((NP, CHUNK, W), jnp.float32),
            pltpu.SemaphoreType.DMA((NZ - 1,)),
            pltpu.SemaphoreType.DMA((NZ - 1,)),
            pltpu.SemaphoreType.DMA((NP - 1,)),
            pltpu.SemaphoreType.DMA((NP - 1,)),
        ],
        compiler_params=pltpu.CompilerParams(collective_id=0),
    )(partial)


# device time: 62029 ns/iter; 2.5841x vs baseline; 1.0286x over previous
import jax
import jax.numpy as jnp
from jax import lax
from jax.experimental import pallas as pl
from jax.experimental.pallas import tpu as pltpu

NZ = 4
NP = 8
CHUNK = 256
F = 4096
W = F // NP
WH = W // 2


def _ring_xy(r):
    rx = jnp.where(r < 4, 0, 1)
    ry = jnp.where(r < 4, r, 7 - r)
    return rx, ry


def kernel(x, dy):
    my_x = lax.axis_index("x")
    my_y = lax.axis_index("y")
    my_r = jnp.where(my_x == 0, my_y, 7 - my_y)

    dy_slice = lax.dynamic_slice(dy, (0, my_r * W), (dy.shape[0], W))
    partial = lax.dot_general(
        x, dy_slice, (((0,), (0,)), ((), ())), preferred_element_type=jnp.float32
    )

    def body(p_ref, out_ref,
             comm0, comm1, ag0, ag1,
             rs_send0, rs_recv0, rs_send1, rs_recv1,
             cw_send0, cw_recv0, ccw_send0, ccw_recv0,
             cw_send1, cw_recv1, ccw_send1, ccw_recv1):
        my_x = lax.axis_index("x")
        my_y = lax.axis_index("y")
        my_z = lax.axis_index("z")
        my_r = jnp.where(my_x == 0, my_y, 7 - my_y)

        z_left = (my_z + NZ - 1) % NZ
        z_right = (my_z + 1) % NZ
        rl_x, rl_y = _ring_xy((my_r + NP - 1) % NP)
        rr_x, rr_y = _ring_xy((my_r + 1) % NP)

        comm = (comm0, comm1)
        ag = (ag0, ag1)
        rs_sems = ((rs_send0, rs_recv0), (rs_send1, rs_recv1))
        cw_sems = ((cw_send0, cw_recv0), (cw_send1, cw_recv1))
        ccw_sems = ((ccw_send0, ccw_recv0), (ccw_send1, ccw_recv1))

        def rs_rdma(k, s):
            send, recv = rs_sems[k]
            return pltpu.make_async_remote_copy(
                src_ref=comm[k].at[s],
                dst_ref=comm[k].at[s + 1],
                send_sem=send.at[s],
                recv_sem=recv.at[s],
                device_id=(my_x, my_y, z_right),
                device_id_type=pl.DeviceIdType.MESH,
            )

        def ag_rdma(k, direction, h):
            if direction == "cw":
                o = (my_r + NP - h) % NP
                send, recv = cw_sems[k]
                dev = (rr_x, rr_y, my_z)
            else:
                o = (my_r + h) % NP
                send, recv = ccw_sems[k]
                dev = (rl_x, rl_y, my_z)
            return pltpu.make_async_remote_copy(
                src_ref=ag[k].at[o],
                dst_ref=ag[k].at[o],
                send_sem=send.at[h],
                recv_sem=recv.at[h],
                device_id=dev,
                device_id_type=pl.DeviceIdType.MESH,
            )

        barrier_sem = pltpu.get_barrier_semaphore()
        for dev in (
            (my_x, my_y, z_left),
            (my_x, my_y, z_right),
            (rl_x, rl_y, my_z),
            (rr_x, rr_y, my_z),
        ):
            pl.semaphore_signal(
                barrier_sem, inc=1, device_id=dev,
                device_id_type=pl.DeviceIdType.MESH,
            )
        pl.semaphore_wait(barrier_sem, 4)

        j0 = (my_z + NZ - 1) % NZ

        def rs_chain(k):
            cols = slice(k * WH, (k + 1) * WH)
            comm[k][0] = p_ref[pl.ds(j0 * CHUNK, CHUNK), cols]
            for s in range(NZ - 1):
                rdma = rs_rdma(k, s)
                rdma.start()
                rdma.wait()
                j = (my_z + 2 * NZ - 2 - s) % NZ
                if s < NZ - 2:
                    comm[k][s + 1] = (
                        comm[k][s + 1] + p_ref[pl.ds(j * CHUNK, CHUNK), cols]
                    )
                else:
                    ag[k][pl.ds(my_r, 1)] = (
                        comm[k][s + 1] + p_ref[pl.ds(j * CHUNK, CHUNK), cols]
                    )[jnp.newaxis]

        n_cw = (4, 3)
        n_ccw = (3, 4)

        def ag_start(k, h):
            started = []
            if h < n_cw[k]:
                r = ag_rdma(k, "cw", h)
                r.start()
                started.append(r)
            if h < n_ccw[k]:
                r = ag_rdma(k, "ccw", h)
                r.start()
                started.append(r)
            return started

        rs_chain(0)
        ag0_h0 = ag_start(0, 0)

        cols1 = slice(WH, 2 * WH)
        comm1[0] = p_ref[pl.ds(j0 * CHUNK, CHUNK), cols1]
        ag0_inflight = ag0_h0
        for s in range(NZ - 1):
            rdma = rs_rdma(1, s)
            rdma.start()
            rdma.wait()
            j = (my_z + 2 * NZ - 2 - s) % NZ
            if s < NZ - 2:
                comm1[s + 1] = comm1[s + 1] + p_ref[pl.ds(j * CHUNK, CHUNK), cols1]
            else:
                ag1[pl.ds(my_r, 1)] = (
                    comm1[s + 1] + p_ref[pl.ds(j * CHUNK, CHUNK), cols1]
                )[jnp.newaxis]
            for r in ag0_inflight:
                r.wait()
            ag0_inflight = ag_start(0, s + 1)

        ag1_inflight = ag_start(1, 0)
        for r in ag0_inflight:
            r.wait()
        for h in range(1, 4):
            for r in ag1_inflight:
                r.wait()
            ag1_inflight = ag_start(1, h)
        for r in ag1_inflight:
            r.wait()

        for j in range(NP):
            out_ref[:, j * W:j * W + WH] = ag0[j]
            out_ref[:, j * W + WH:(j + 1) * W] = ag1[j]

    return pl.pallas_call(
        body,
        out_shape=jax.ShapeDtypeStruct((CHUNK, F), jnp.float32),
        in_specs=[pl.BlockSpec(memory_space=pltpu.VMEM)],
        out_specs=pl.BlockSpec(memory_space=pltpu.VMEM),
        scratch_shapes=[
            pltpu.VMEM((NZ, CHUNK, WH), jnp.float32),
            pltpu.VMEM((NZ, CHUNK, WH), jnp.float32),
            pltpu.VMEM((NP, CHUNK, WH), jnp.float32),
            pltpu.VMEM((NP, CHUNK, WH), jnp.float32),
            pltpu.SemaphoreType.DMA((NZ - 1,)),
            pltpu.SemaphoreType.DMA((NZ - 1,)),
            pltpu.SemaphoreType.DMA((NZ - 1,)),
            pltpu.SemaphoreType.DMA((NZ - 1,)),
            pltpu.SemaphoreType.DMA((4,)),
            pltpu.SemaphoreType.DMA((4,)),
            pltpu.SemaphoreType.DMA((4,)),
            pltpu.SemaphoreType.DMA((4,)),
            pltpu.SemaphoreType.DMA((4,)),
            pltpu.SemaphoreType.DMA((4,)),
            pltpu.SemaphoreType.DMA((4,)),
            pltpu.SemaphoreType.DMA((4,)),
        ],
        compiler_params=pltpu.CompilerParams(collective_id=0),
    )(partial)


# device time: 48698 ns/iter; 3.2916x vs baseline; 1.2737x over previous
import jax
import jax.numpy as jnp
from jax import lax
from jax.experimental import pallas as pl
from jax.experimental.pallas import tpu as pltpu

NZ = 4
NP = 8
CHUNK = 256
F = 4096
W = F // NP
K = 4
WS = W // K


def _ring_xy(r):
    rx = jnp.where(r < 4, 0, 1)
    ry = jnp.where(r < 4, r, 7 - r)
    return rx, ry


def kernel(x, dy):
    my_x = lax.axis_index("x")
    my_y = lax.axis_index("y")
    my_r = jnp.where(my_x == 0, my_y, 7 - my_y)

    dy_slice = lax.dynamic_slice(dy, (0, my_r * W), (dy.shape[0], W))
    partial = lax.dot_general(
        x, dy_slice, (((0,), (0,)), ((), ())), preferred_element_type=jnp.float32
    )

    def body(p_ref, out_ref, comm, ag,
             rs_send, rs_recv, cw_send, cw_recv, ccw_send, ccw_recv):
        my_x = lax.axis_index("x")
        my_y = lax.axis_index("y")
        my_z = lax.axis_index("z")
        my_r = jnp.where(my_x == 0, my_y, 7 - my_y)

        z_left = (my_z + NZ - 1) % NZ
        z_right = (my_z + 1) % NZ
        rl_x, rl_y = _ring_xy((my_r + NP - 1) % NP)
        rr_x, rr_y = _ring_xy((my_r + 1) % NP)

        def n_cw(k):
            return 4 if k % 2 == 0 else 3

        def n_ccw(k):
            return 3 if k % 2 == 0 else 4

        def rs_rdma(k, s):
            return pltpu.make_async_remote_copy(
                src_ref=comm.at[k, s],
                dst_ref=comm.at[k, s + 1],
                send_sem=rs_send.at[k, s],
                recv_sem=rs_recv.at[k, s],
                device_id=(my_x, my_y, z_right),
                device_id_type=pl.DeviceIdType.MESH,
            )

        def ag_rdma(k, direction, h):
            if direction == "cw":
                o = (my_r + NP - h) % NP
                send, recv = cw_send, cw_recv
                dev = (rr_x, rr_y, my_z)
            else:
                o = (my_r + h) % NP
                send, recv = ccw_send, ccw_recv
                dev = (rl_x, rl_y, my_z)
            return pltpu.make_async_remote_copy(
                src_ref=ag.at[k, o],
                dst_ref=ag.at[k, o],
                send_sem=send.at[k, h],
                recv_sem=recv.at[k, h],
                device_id=dev,
                device_id_type=pl.DeviceIdType.MESH,
            )

        def ag_start(k, h):
            started = []
            if h < n_cw(k):
                r = ag_rdma(k, "cw", h)
                r.start()
                started.append(r)
            if h < n_ccw(k):
                r = ag_rdma(k, "ccw", h)
                r.start()
                started.append(r)
            return started

        barrier_sem = pltpu.get_barrier_semaphore()
        for dev in (
            (my_x, my_y, z_left),
            (my_x, my_y, z_right),
            (rl_x, rl_y, my_z),
            (rr_x, rr_y, my_z),
        ):
            pl.semaphore_signal(
                barrier_sem, inc=1, device_id=dev,
                device_id_type=pl.DeviceIdType.MESH,
            )
        pl.semaphore_wait(barrier_sem, 4)

        j0 = (my_z + NZ - 1) % NZ

        rs_live = {}
        for k in range(K):
            comm[k, 0] = p_ref[pl.ds(j0 * CHUNK, CHUNK), k * WS:(k + 1) * WS]
        for k in range(K):
            r = rs_rdma(k, 0)
            r.start()
            rs_live[(k, 0)] = r

        ag_live = {}
        for s in range(NZ - 1):
            for k in range(K):
                rs_live[(k, s)].wait()
                j = (my_z + 2 * NZ - 2 - s) % NZ
                cols = slice(k * WS, (k + 1) * WS)
                if s < NZ - 2:
                    comm[k, s + 1] = (
                        comm[k, s + 1] + p_ref[pl.ds(j * CHUNK, CHUNK), cols]
                    )
                    r = rs_rdma(k, s + 1)
                    r.start()
                    rs_live[(k, s + 1)] = r
                else:
                    ag[k, pl.ds(my_r, 1)] = (
                        comm[k, s + 1] + p_ref[pl.ds(j * CHUNK, CHUNK), cols]
                    )[jnp.newaxis]
                    ag_live[(k, 0)] = ag_start(k, 0)

        for h in range(4):
            for k in range(K):
                for r in ag_live[(k, h)]:
                    r.wait()
                if h < 3:
                    ag_live[(k, h + 1)] = ag_start(k, h + 1)
                else:
                    for j in range(NP):
                        out_ref[:, j * W + k * WS:j * W + (k + 1) * WS] = ag[k, j]

    return pl.pallas_call(
        body,
        out_shape=jax.ShapeDtypeStruct((CHUNK, F), jnp.float32),
        in_specs=[pl.BlockSpec(memory_space=pltpu.VMEM)],
        out_specs=pl.BlockSpec(memory_space=pltpu.VMEM),
        scratch_shapes=[
            pltpu.VMEM((K, NZ, CHUNK, WS), jnp.float32),
            pltpu.VMEM((K, NP, CHUNK, WS), jnp.float32),
            pltpu.SemaphoreType.DMA((K, NZ - 1)),
            pltpu.SemaphoreType.DMA((K, NZ - 1)),
            pltpu.SemaphoreType.DMA((K, 4)),
            pltpu.SemaphoreType.DMA((K, 4)),
            pltpu.SemaphoreType.DMA((K, 4)),
            pltpu.SemaphoreType.DMA((K, 4)),
        ],
        compiler_params=pltpu.CompilerParams(collective_id=0),
    )(partial)
